# baseline (speedup 1.0000x reference)
import functools

import jax
import jax.numpy as jnp
from jax import lax
from jax.experimental import pallas as pl
from jax.experimental.pallas import tpu as pltpu
from jax.experimental.pallas import tpu_sc as plsc

_NUM_CORES = 2
_NUM_SUBCORES = 16
_NUM_WORKERS = _NUM_CORES * _NUM_SUBCORES
_CHUNK = 320
_N_CHUNKS = 1


def _ln_table_body(table_ref, gamma_ref, beta_ref, out_ref):
    t = table_ref[...]
    mu = jnp.mean(t, axis=1, keepdims=True)
    var = jnp.mean((t - mu) ** 2, axis=1, keepdims=True)
    out_ref[...] = (t - mu) * lax.rsqrt(var + 1e-5) * gamma_ref[...] + beta_ref[...]


def _sc_gather(norm_table, idx_flat):
    n, d = idx_flat.shape[0], norm_table.shape[1]
    v = norm_table.shape[0]
    per_w = n // _NUM_WORKERS
    chunk = _CHUNK
    while per_w % chunk:
        chunk -= 8
    n_it = per_w // chunk
    mesh = plsc.VectorSubcoreMesh(
        core_axis_name="c", subcore_axis_name="s",
        num_cores=_NUM_CORES, num_subcores=_NUM_SUBCORES)

    @functools.partial(
        pl.kernel,
        mesh=mesh,
        out_type=jax.ShapeDtypeStruct((n, d), jnp.float32),
        scratch_types=[
            pltpu.VMEM_SHARED((v, d), jnp.float32),
            pltpu.VMEM((per_w,), jnp.int32),
            pltpu.VMEM((chunk, d), jnp.float32),
            pltpu.VMEM((chunk, d), jnp.float32),
            pltpu.SemaphoreType.DMA,
            pltpu.SemaphoreType.DMA,
        ],
        compiler_params=pltpu.CompilerParams(use_tc_tiling_on_sc=False),
    )
    def gather_k(tbl_hbm, idx_hbm, out_hbm, tbl_v, idx_v, rows0, rows1,
                 sem0, sem1):
        wid = lax.axis_index("s") * _NUM_CORES + lax.axis_index("c")
        base = wid * per_w

        @pl.when(lax.axis_index("s") == 0)
        def _():
            pltpu.sync_copy(tbl_hbm, tbl_v)

        pltpu.sync_copy(idx_hbm.at[pl.ds(base, per_w)], idx_v)
        plsc.subcore_barrier()

        def start(j, rows, sem):
            pltpu.async_copy(
                tbl_v.at[idx_v.at[pl.ds(j * chunk, chunk)]], rows, sem)

        def drain(j, rows, sem):
            pltpu.make_async_copy(
                tbl_v.at[idx_v.at[pl.ds(j * chunk, chunk)]], rows, sem
            ).wait()
            pltpu.sync_copy(rows, out_hbm.at[pl.ds(base + j * chunk, chunk)])

        start(0, rows0, sem0)

        def body(k, carry):
            j0 = 2 * k
            start(j0 + 1, rows1, sem1)
            drain(j0, rows0, sem0)

            @pl.when(j0 + 2 < n_it)
            def _():
                start(j0 + 2, rows0, sem0)

            drain(j0 + 1, rows1, sem1)
            return carry

        lax.fori_loop(0, n_it // 2, body, 0)
        if n_it % 2:
            drain(n_it - 1, rows0, sem0)

    return gather_k(norm_table, idx_flat)


def _lstm_body(emb_ref, wpp_ref, bias_ref, hout_ref, c_scr):
    t = pl.program_id(0)
    live = t > 0

    xt_pp = jnp.swapaxes(emb_ref[0], 0, 1)
    hp = jnp.where(live, hout_ref[...], 0.0)
    cp = jnp.where(live, c_scr[...], 0.0)
    cat_pp = jnp.concatenate([xt_pp, hp], axis=0)
    gates = jnp.dot(wpp_ref[...], cat_pp,
                    preferred_element_type=jnp.float32) + bias_ref[...]
    h2 = c_scr.shape[0]
    i = 0.5 * jnp.tanh(gates[0 * h2:1 * h2]) + 0.5
    f = 0.5 * jnp.tanh(gates[1 * h2:2 * h2]) + 0.5
    g = jnp.tanh(gates[2 * h2:3 * h2])
    o = 0.5 * jnp.tanh(gates[3 * h2:4 * h2]) + 0.5
    c = f * cp + i * g
    hout_ref[...] = o * jnp.tanh(c)
    c_scr[...] = c


def _cls_body(h_ref, wcls_ref, bcls_ref, out_ref):
    out_ref[...] = lax.dot_general(
        h_ref[...], wcls_ref[...], (((0,), (0,)), ((), ())),
        preferred_element_type=jnp.float32) + bcls_ref[...]


def kernel(x, table, gamma, beta, W_ih, W_hh, b_ih, b_hh, W_cls, b_cls):
    B, T = x.shape
    V, D = table.shape
    H = W_hh.shape[1]
    C = W_cls.shape[0]

    norm_table = pl.pallas_call(
        _ln_table_body,
        out_shape=jax.ShapeDtypeStruct((V, D), jnp.float32),
    )(table, gamma.reshape(1, D), beta.reshape(1, D))

    z_d = jnp.zeros((H, D), jnp.float32)
    z_h = jnp.zeros((H, H), jnp.float32)
    blocks = []
    for gi in range(4):
        wi = W_ih[gi * H:(gi + 1) * H]
        wh = W_hh[gi * H:(gi + 1) * H]
        blocks.append(jnp.concatenate([wi, z_d, wh, z_h], axis=1))
        blocks.append(jnp.concatenate([z_d, wi, z_h, wh], axis=1))
    wpp = jnp.concatenate(blocks, axis=0)
    bias4 = (b_ih + b_hh)
    bias_pp = jnp.concatenate(
        [jnp.tile(bias4[gi * H:(gi + 1) * H], 2) for gi in range(4)]
    ).reshape(8 * H, 1)
    gate_scale = jnp.where(
        (jnp.arange(8 * H) >= 4 * H) & (jnp.arange(8 * H) < 6 * H), 1.0, 0.5
    ).astype(jnp.float32)
    wpp = wpp * gate_scale[:, None]
    bias_pp = bias_pp * gate_scale[:, None]

    z_hc = jnp.zeros((H, C), jnp.float32)
    wcls_pp = jnp.concatenate([
        jnp.concatenate([W_cls.T, z_hc], axis=1),
        jnp.concatenate([z_hc, W_cls.T], axis=1)], axis=0)
    bcls_pp = jnp.tile(b_cls, 2).reshape(1, 2 * C)

    bc = B // _N_CHUNKS
    bp = bc // 2
    lstm = pl.pallas_call(
        _lstm_body,
        grid=(T,),
        in_specs=[
            pl.BlockSpec((1, bp, 2 * D), lambda t: (t, 0, 0)),
            pl.BlockSpec((8 * H, 2 * (D + H)), lambda t: (0, 0)),
            pl.BlockSpec((8 * H, 1), lambda t: (0, 0)),
        ],
        out_specs=pl.BlockSpec((2 * H, bp), lambda t: (0, 0)),
        out_shape=jax.ShapeDtypeStruct((2 * H, bp), jnp.float32),
        scratch_shapes=[
            pltpu.VMEM((2 * H, bp), jnp.float32),
        ],
    )
    classify = pl.pallas_call(
        _cls_body,
        out_shape=jax.ShapeDtypeStruct((bp, 2 * C), jnp.float32),
    )

    outs = []
    for k in range(_N_CHUNKS):
        xk = lax.slice_in_dim(x, k * bc, (k + 1) * bc, axis=0)
        idx_k = xk.T.reshape(-1)
        emb_k = _sc_gather(norm_table, idx_k)
        emb_k = emb_k.reshape(T, bp, 2 * D)
        h_k = lstm(emb_k, wpp, bias_pp)
        outs.append(classify(h_k, wcls_pp, bcls_pp))
    return jnp.concatenate([o.reshape(bc, C) for o in outs], axis=0)


# speedup vs baseline: 1.0225x; 1.0225x over previous
import functools

import jax
import jax.numpy as jnp
from jax import lax
from jax.experimental import pallas as pl
from jax.experimental.pallas import tpu as pltpu
from jax.experimental.pallas import tpu_sc as plsc

_NUM_CORES = 2
_NUM_SUBCORES = 16
_NUM_WORKERS = _NUM_CORES * _NUM_SUBCORES
_CHUNK = 320
_N_CHUNKS = 1


def _ln_table_body(table_ref, gamma_ref, beta_ref, out_ref):
    t = table_ref[...]
    mu = jnp.mean(t, axis=1, keepdims=True)
    var = jnp.mean((t - mu) ** 2, axis=1, keepdims=True)
    out_ref[...] = (t - mu) * lax.rsqrt(var + 1e-5) * gamma_ref[...] + beta_ref[...]


def _sc_gather(norm_table, idx_flat):
    n, d = idx_flat.shape[0], norm_table.shape[1]
    v = norm_table.shape[0]
    per_w = n // _NUM_WORKERS
    chunk = _CHUNK
    while per_w % chunk:
        chunk -= 8
    n_it = per_w // chunk
    mesh = plsc.VectorSubcoreMesh(
        core_axis_name="c", subcore_axis_name="s",
        num_cores=_NUM_CORES, num_subcores=_NUM_SUBCORES)

    @functools.partial(
        pl.kernel,
        mesh=mesh,
        out_type=jax.ShapeDtypeStruct((n, d), jnp.float32),
        scratch_types=[
            pltpu.VMEM_SHARED((v, d), jnp.float32),
            pltpu.VMEM((per_w,), jnp.int32),
            pltpu.VMEM((chunk, d), jnp.float32),
            pltpu.VMEM((chunk, d), jnp.float32),
            pltpu.SemaphoreType.DMA,
            pltpu.SemaphoreType.DMA,
        ],
        compiler_params=pltpu.CompilerParams(use_tc_tiling_on_sc=False),
    )
    def gather_k(tbl_hbm, idx_hbm, out_hbm, tbl_v, idx_v, rows0, rows1,
                 sem0, sem1):
        wid = lax.axis_index("s") * _NUM_CORES + lax.axis_index("c")
        base = wid * per_w

        @pl.when(lax.axis_index("s") == 0)
        def _():
            pltpu.sync_copy(tbl_hbm, tbl_v)

        pltpu.sync_copy(idx_hbm.at[pl.ds(base, per_w)], idx_v)
        plsc.subcore_barrier()

        def start(j, rows, sem):
            pltpu.async_copy(
                tbl_v.at[idx_v.at[pl.ds(j * chunk, chunk)]], rows, sem)

        def drain(j, rows, sem):
            pltpu.make_async_copy(
                tbl_v.at[idx_v.at[pl.ds(j * chunk, chunk)]], rows, sem
            ).wait()
            pltpu.sync_copy(rows, out_hbm.at[pl.ds(base + j * chunk, chunk)])

        start(0, rows0, sem0)

        def body(k, carry):
            j0 = 2 * k
            start(j0 + 1, rows1, sem1)
            drain(j0, rows0, sem0)

            @pl.when(j0 + 2 < n_it)
            def _():
                start(j0 + 2, rows0, sem0)

            drain(j0 + 1, rows1, sem1)
            return carry

        lax.fori_loop(0, n_it // 2, body, 0)
        if n_it % 2:
            drain(n_it - 1, rows0, sem0)

    return gather_k(norm_table, idx_flat)


def _lstm_body(emb_ref, wpp_ref, bias_ref, wcls_ref, bcls_ref, out_ref,
               h_scr, c_scr):
    t = pl.program_id(0)
    nt = pl.num_programs(0)
    h2 = h_scr.shape[0]

    live = t > 0
    xt_pp = jnp.swapaxes(emb_ref[0], 0, 1)
    hp = jnp.where(live, h_scr[...], 0.0)
    cp = jnp.where(live, c_scr[...], 0.0)
    cat_pp = jnp.concatenate([xt_pp, hp], axis=0)
    gates = jnp.dot(wpp_ref[...], cat_pp,
                    preferred_element_type=jnp.float32) + bias_ref[...]
    i = 0.5 * jnp.tanh(gates[0 * h2:1 * h2]) + 0.5
    f = 0.5 * jnp.tanh(gates[1 * h2:2 * h2]) + 0.5
    g = jnp.tanh(gates[2 * h2:3 * h2])
    o = 0.5 * jnp.tanh(gates[3 * h2:4 * h2]) + 0.5
    c = f * cp + i * g
    hn = o * jnp.tanh(c)
    h_scr[...] = hn
    c_scr[...] = c

    @pl.when(t == nt - 1)
    def _():
        out_ref[...] = lax.dot_general(
            hn, wcls_ref[...], (((0,), (0,)), ((), ())),
            preferred_element_type=jnp.float32) + bcls_ref[...]


def kernel(x, table, gamma, beta, W_ih, W_hh, b_ih, b_hh, W_cls, b_cls):
    B, T = x.shape
    V, D = table.shape
    H = W_hh.shape[1]
    C = W_cls.shape[0]

    norm_table = pl.pallas_call(
        _ln_table_body,
        out_shape=jax.ShapeDtypeStruct((V, D), jnp.float32),
    )(table, gamma.reshape(1, D), beta.reshape(1, D))

    z_d = jnp.zeros((H, D), jnp.float32)
    z_h = jnp.zeros((H, H), jnp.float32)
    blocks = []
    for gi in range(4):
        wi = W_ih[gi * H:(gi + 1) * H]
        wh = W_hh[gi * H:(gi + 1) * H]
        blocks.append(jnp.concatenate([wi, z_d, wh, z_h], axis=1))
        blocks.append(jnp.concatenate([z_d, wi, z_h, wh], axis=1))
    wpp = jnp.concatenate(blocks, axis=0)
    bias4 = (b_ih + b_hh)
    bias_pp = jnp.concatenate(
        [jnp.tile(bias4[gi * H:(gi + 1) * H], 2) for gi in range(4)]
    ).reshape(8 * H, 1)
    gate_scale = jnp.where(
        (jnp.arange(8 * H) >= 4 * H) & (jnp.arange(8 * H) < 6 * H), 1.0, 0.5
    ).astype(jnp.float32)
    wpp = wpp * gate_scale[:, None]
    bias_pp = bias_pp * gate_scale[:, None]

    z_hc = jnp.zeros((H, C), jnp.float32)
    wcls_pp = jnp.concatenate([
        jnp.concatenate([W_cls.T, z_hc], axis=1),
        jnp.concatenate([z_hc, W_cls.T], axis=1)], axis=0)
    bcls_pp = jnp.tile(b_cls, 2).reshape(1, 2 * C)

    bc = B // _N_CHUNKS
    bp = bc // 2
    lstm = pl.pallas_call(
        _lstm_body,
        grid=(T,),
        in_specs=[
            pl.BlockSpec((1, bp, 2 * D), lambda t: (t, 0, 0)),
            pl.BlockSpec((8 * H, 2 * (D + H)), lambda t: (0, 0)),
            pl.BlockSpec((8 * H, 1), lambda t: (0, 0)),
            pl.BlockSpec((2 * H, 2 * C), lambda t: (0, 0)),
            pl.BlockSpec((1, 2 * C), lambda t: (0, 0)),
        ],
        out_specs=pl.BlockSpec((bp, 2 * C), lambda t: (0, 0)),
        out_shape=jax.ShapeDtypeStruct((bp, 2 * C), jnp.float32),
        scratch_shapes=[
            pltpu.VMEM((2 * H, bp), jnp.float32),
            pltpu.VMEM((2 * H, bp), jnp.float32),
        ],
    )

    outs = []
    for k in range(_N_CHUNKS):
        xk = lax.slice_in_dim(x, k * bc, (k + 1) * bc, axis=0)
        idx_k = xk.T.reshape(-1)
        emb_k = _sc_gather(norm_table, idx_k)
        emb_k = emb_k.reshape(T, bp, 2 * D)
        outs.append(lstm(emb_k, wpp, bias_pp, wcls_pp, bcls_pp))
    return jnp.concatenate([o.reshape(bc, C) for o in outs], axis=0)
